# Initial kernel scaffold; baseline (speedup 1.0000x reference)
import jax
import jax.numpy as jnp
from jax.experimental import pallas as pl


def kernel(x, edge_index, edge_weight, W1, b1, W2, b2):
    raise NotImplementedError("write your pallas kernel here")


# speedup vs baseline: 5.2093x; 5.2093x over previous
import functools

import jax
import jax.numpy as jnp
from jax import lax
from jax.experimental import pallas as pl
from jax.experimental.pallas import tpu as pltpu
from jax.experimental.pallas import tpu_sc as plsc

N = 10000
E = 320000
D = 128
L = 16
GROUP = 128
G = E // GROUP
NC = 2
NS = 16
G_PER_SC = G // NC
G_TILE = G_PER_SC // NS
G_REM = G_PER_SC - G_TILE * NS
PCHUNK = 80
NPC = N // PCHUNK
PC_TILE = NPC // NS
PC_REM = NPC - PC_TILE * NS
MM_BLK = 2000


def _sc_layer(h, src, dst, w):
    mesh = plsc.VectorSubcoreMesh(core_axis_name="c", subcore_axis_name="s")

    @functools.partial(
        pl.kernel,
        out_type=jax.ShapeDtypeStruct((NC, N, D), jnp.float32),
        mesh=mesh,
        scratch_types=[
            pltpu.VMEM_SHARED((N, D), jnp.float32),
            pltpu.VMEM((GROUP,), jnp.int32),
            pltpu.VMEM((GROUP,), jnp.int32),
            pltpu.VMEM((GROUP,), jnp.float32),
            pltpu.VMEM((GROUP, D), jnp.float32),
            pltpu.VMEM((PCHUNK, D), jnp.float32),
            pltpu.SemaphoreType.DMA,
        ],
    )
    def sc_kernel(h_hbm, src_hbm, dst_hbm, w_hbm, out_hbm,
                  acc, src_v, dst_v, w_v, rows_v, stage_v, sem):
        c = lax.axis_index("c")
        s = lax.axis_index("s")

        def _zero(r, carry):
            for j in range(D // L):
                stage_v[r, pl.ds(j * L, L)] = jnp.zeros((L,), jnp.float32)
            return carry
        lax.fori_loop(0, PCHUNK, _zero, 0)
        pc0 = s * PC_TILE + jnp.minimum(s, PC_REM)
        pcnt = PC_TILE + jnp.where(s < PC_REM, 1, 0)

        def _zacc(k, carry):
            pltpu.sync_copy(stage_v, acc.at[pl.ds((pc0 + k) * PCHUNK, PCHUNK)])
            return carry
        lax.fori_loop(0, pcnt, _zacc, 0)
        plsc.subcore_barrier()

        g0 = c * G_PER_SC + s * G_TILE + jnp.minimum(s, G_REM)
        cnt = G_TILE + jnp.where(s < G_REM, 1, 0)

        def body(i, carry):
            g = g0 + i
            pltpu.sync_copy(src_hbm.at[g], src_v)
            pltpu.sync_copy(dst_hbm.at[g], dst_v)
            pltpu.sync_copy(w_hbm.at[g], w_v)
            pltpu.async_copy(h_hbm.at[src_v], rows_v, sem).wait()

            def scale(eb, inner):
                wv16 = w_v[pl.ds(eb * L, L)]
                for lane in range(L):
                    wv = jnp.full((L,), wv16[lane], jnp.float32)
                    e = eb * L + lane
                    for j in range(D // L):
                        rows_v[e, pl.ds(j * L, L)] = (
                            rows_v[e, pl.ds(j * L, L)] * wv)
                return inner
            lax.fori_loop(0, GROUP // L, scale, 0)

            pltpu.sync_copy(rows_v, acc.at[dst_v], add=True)
            return carry
        lax.fori_loop(0, cnt, body, 0)
        plsc.subcore_barrier()

        def _pub(k, carry):
            r0 = (pc0 + k) * PCHUNK
            pltpu.sync_copy(acc.at[pl.ds(r0, PCHUNK)], stage_v)
            pltpu.sync_copy(stage_v, out_hbm.at[c, pl.ds(r0, PCHUNK)])
            return carry
        lax.fori_loop(0, pcnt, _pub, 0)

    return sc_kernel(h, src, dst, w)


def _mm_fused(p, b, W):
    def body(p_ref, b_ref, w_ref, o_ref):
        hs = p_ref[0] + p_ref[1]
        o_ref[...] = jnp.dot(hs, w_ref[...],
                             preferred_element_type=jnp.float32) + b_ref[...]
    return pl.pallas_call(
        body,
        grid=(N // MM_BLK,),
        in_specs=[pl.BlockSpec((NC, MM_BLK, D), lambda i: (0, i, 0)),
                  pl.BlockSpec((1, D), lambda i: (0, 0)),
                  pl.BlockSpec((D, D), lambda i: (0, 0))],
        out_specs=pl.BlockSpec((MM_BLK, D), lambda i: (i, 0)),
        out_shape=jax.ShapeDtypeStruct((N, D), jnp.float32),
    )(p, b, W)


def kernel(x, edge_index, edge_weight, W1, b1, W2, b2):
    src = edge_index[0].reshape(G, GROUP)
    dst = edge_index[1].reshape(G, GROUP)
    w = edge_weight.reshape(G, GROUP)
    b1r = b1.reshape(1, D)
    b2r = b2.reshape(1, D)

    p1 = _sc_layer(x, src, dst, w)
    h1 = _mm_fused(p1, b1r, W1)
    p2 = _sc_layer(h1, src, dst, w)
    return _mm_fused(p2, b2r, W2)
